# baseline (speedup 1.0000x reference)
import functools

import jax
import jax.numpy as jnp
from jax import lax
from jax.experimental import pallas as pl
from jax.experimental.pallas import tpu as pltpu
from jax.experimental.pallas import tpu_sc as plsc

E = 8
K = 2
D = 1024
H = 1024
T = 2048
TK = T * K
BK = 256
R = 6144
NB = R // BK

NC = 2
NS = 16
NW = NC * NS


def _router_body(x_ref, rw_ref, i1_ref, i2_ref, g1_ref, g2_ref):
    x = x_ref[...]
    rw = rw_ref[...]
    logits = lax.dot_general(x, rw, (((1,), (1,)), ((), ())),
                             preferred_element_type=jnp.float32)
    cols = lax.broadcasted_iota(jnp.int32, logits.shape, 1)
    i1 = jnp.argmax(logits, axis=1).astype(jnp.int32)
    m1 = jnp.max(logits, axis=1)
    masked = jnp.where(cols == i1[:, None], -jnp.inf, logits)
    i2 = jnp.argmax(masked, axis=1).astype(jnp.int32)
    m2 = jnp.max(masked, axis=1)
    g1 = 1.0 / (1.0 + jnp.exp(m2 - m1))
    i1_ref[...] = i1
    i2_ref[...] = i2
    g1_ref[...] = g1
    g2_ref[...] = 1.0 - g1


def _router(x, router_w):
    return pl.pallas_call(
        _router_body,
        out_shape=(
            jax.ShapeDtypeStruct((T,), jnp.int32),
            jax.ShapeDtypeStruct((T,), jnp.int32),
            jax.ShapeDtypeStruct((T,), jnp.float32),
            jax.ShapeDtypeStruct((T,), jnp.float32),
        ),
    )(x, router_w)


_XTW = T // NW
_XCH = 32
_XNC = _XTW // _XCH


def _dispatch_body(x_hbm, p0_hbm, p1_hbm, xs_hbm, i0_v, i1_v, buf0_v, buf1_v,
                   g0, g1, s0, s1):
    wid = lax.axis_index("s") * NC + lax.axis_index("c")
    base = wid * _XTW
    pltpu.sync_copy(p0_hbm.at[pl.ds(wid * _XNC, _XNC)], i0_v)
    pltpu.sync_copy(p1_hbm.at[pl.ds(wid * _XNC, _XNC)], i1_v)
    bufs = (buf0_v, buf1_v)
    gsem = (g0, g1)
    ssem = (s0, s1)
    rd = [pltpu.make_async_copy(x_hbm.at[pl.ds(base + j * _XCH, _XCH)],
                                bufs[j], gsem[j])
          for j in range(_XNC)]
    w0 = [pltpu.make_async_copy(bufs[j], xs_hbm.at[i0_v.at[j]], ssem[j])
          for j in range(_XNC)]
    w1 = [pltpu.make_async_copy(bufs[j], xs_hbm.at[i1_v.at[j]], ssem[j])
          for j in range(_XNC)]
    for j in range(_XNC):
        rd[j].start()
    for j in range(_XNC):
        rd[j].wait()
        w0[j].start()
        w1[j].start()
    for j in range(_XNC):
        w0[j].wait()
        w1[j].wait()


def _dispatch(x, p0, p1):
    mesh = plsc.VectorSubcoreMesh(core_axis_name="c", subcore_axis_name="s")
    return pl.kernel(
        _dispatch_body,
        out_type=jax.ShapeDtypeStruct((R, D), jnp.float32),
        mesh=mesh,
        scratch_types=[
            pltpu.VMEM((_XNC, _XCH), jnp.int32),
            pltpu.VMEM((_XNC, _XCH), jnp.int32),
            pltpu.VMEM((_XCH, D), jnp.float32),
            pltpu.VMEM((_XCH, D), jnp.float32),
            pltpu.SemaphoreType.DMA,
            pltpu.SemaphoreType.DMA,
            pltpu.SemaphoreType.DMA,
            pltpu.SemaphoreType.DMA,
        ],
    )(x, p0, p1)


_KS = 8
_HC = 2 * H // _KS


def _mlp_body(be_ref, x_ref, wi_ref, wo_ref, o_ref, h_acc):
    k = pl.program_id(1)
    x = x_ref[...]
    part = lax.dot_general(x, wi_ref[0], (((1,), (1,)), ((), ())),
                           precision=lax.Precision.DEFAULT,
                           preferred_element_type=jnp.float32)
    h_acc[:, pl.ds(k * _HC, _HC)] = part

    @pl.when(k == _KS - 1)
    def _():
        h = h_acc[...]
        left = h[:, :H]
        right = h[:, H:]
        a = left * jax.nn.sigmoid(left) * right
        wo = wo_ref[0]
        y = lax.dot_general(a.astype(jnp.bfloat16), wo, (((1,), (1,)), ((), ())),
                            precision=lax.Precision.DEFAULT,
                            preferred_element_type=jnp.float32)
        o_ref[...] = y


def _mlp(blk_expert, xs, w_in, w_out_bf):
    grid_spec = pltpu.PrefetchScalarGridSpec(
        num_scalar_prefetch=1,
        grid=(NB, _KS),
        in_specs=[
            pl.BlockSpec((BK, D), lambda b, k, be: (b, 0)),
            pl.BlockSpec((1, _HC, D), lambda b, k, be: (be[b], k, 0)),
            pl.BlockSpec((1, D, H), lambda b, k, be: (be[b], 0, 0)),
        ],
        out_specs=pl.BlockSpec((BK, D), lambda b, k, be: (b, 0)),
        scratch_shapes=[pltpu.VMEM((BK, 2 * H), jnp.float32)],
    )
    return pl.pallas_call(
        _mlp_body,
        grid_spec=grid_spec,
        out_shape=jax.ShapeDtypeStruct((R, D), jnp.float32),
        compiler_params=pltpu.CompilerParams(
            dimension_semantics=("arbitrary", "arbitrary")),
    )(blk_expert, xs, w_in, w_out_bf)


_TPW = T // NW
_CCH = 16
_CNC = _TPW // _CCH


def _combine_body(ys_hbm, p0_hbm, p1_hbm, g0b_hbm, g1b_hbm, out_hbm, i0_v, i1_v,
                  g0v, g1v, a0_v, b0_v, a1_v, b1_v, g0, g1, s0, s1):
    wid = lax.axis_index("s") * NC + lax.axis_index("c")
    base = wid * _TPW
    pltpu.sync_copy(p0_hbm.at[pl.ds(wid * _CNC, _CNC)], i0_v)
    pltpu.sync_copy(p1_hbm.at[pl.ds(wid * _CNC, _CNC)], i1_v)
    pltpu.sync_copy(g0b_hbm.at[pl.ds(base, _TPW)], g0v)
    pltpu.sync_copy(g1b_hbm.at[pl.ds(base, _TPW)], g1v)
    av = (a0_v, a1_v)
    bv = (b0_v, b1_v)
    gsem = (g0, g1)
    ssem = (s0, s1)
    ga = [pltpu.make_async_copy(ys_hbm.at[i0_v.at[j]], av[j & 1], gsem[j & 1])
          for j in range(_CNC)]
    gb = [pltpu.make_async_copy(ys_hbm.at[i1_v.at[j]], bv[j & 1], gsem[j & 1])
          for j in range(_CNC)]
    scp = [pltpu.make_async_copy(av[j & 1],
                                 out_hbm.at[pl.ds(base + j * _CCH, _CCH)],
                                 ssem[j & 1])
           for j in range(_CNC)]
    ga[0].start()
    gb[0].start()
    for j in range(_CNC):
        if j + 1 < _CNC:
            if j >= 1:
                scp[j - 1].wait()
            ga[j + 1].start()
            gb[j + 1].start()
        ga[j].wait()
        gb[j].wait()
        a_v = av[j & 1]
        b_v = bv[j & 1]

        jj = j * _CCH

        @plsc.parallel_loop(0, _CCH * D, 16, unroll=8)
        def _add(f):
            row = lax.shift_right_logical(f, 10)
            col = pl.multiple_of(lax.bitwise_and(f, D - 1), 16)
            sl = pl.ds(col, 16)
            tr = jj + row
            a_v[row, sl] = (a_v[row, sl] * g0v[tr, :]
                            + b_v[row, sl] * g1v[tr, :])

        scp[j].start()
    scp[_CNC - 2].wait()
    scp[_CNC - 1].wait()


def _combine(ys, p0, p1, g0b, g1b):
    mesh = plsc.VectorSubcoreMesh(core_axis_name="c", subcore_axis_name="s")
    return pl.kernel(
        _combine_body,
        out_type=jax.ShapeDtypeStruct((T, D), jnp.float32),
        mesh=mesh,
        scratch_types=[
            pltpu.VMEM((_CNC, _CCH), jnp.int32),
            pltpu.VMEM((_CNC, _CCH), jnp.int32),
            pltpu.VMEM((_TPW, 16), jnp.float32),
            pltpu.VMEM((_TPW, 16), jnp.float32),
            pltpu.VMEM((_CCH, D), jnp.float32),
            pltpu.VMEM((_CCH, D), jnp.float32),
            pltpu.VMEM((_CCH, D), jnp.float32),
            pltpu.VMEM((_CCH, D), jnp.float32),
            pltpu.SemaphoreType.DMA,
            pltpu.SemaphoreType.DMA,
            pltpu.SemaphoreType.DMA,
            pltpu.SemaphoreType.DMA,
        ],
    )(ys, p0, p1, g0b, g1b)


def kernel(layer_input, router_w, w_in, w_out):
    x = layer_input.reshape(T, D)
    i1, i2, g1, g2 = _router(x, router_w)

    e_flat = jnp.stack([i1, i2], axis=1).reshape(-1)
    oh = (e_flat[:, None] == jnp.arange(E, dtype=jnp.int32)[None, :]).astype(jnp.int32)
    csum = jnp.cumsum(oh, axis=0)
    counts = csum[-1]
    rank = jnp.take_along_axis(csum, e_flat[:, None], axis=1)[:, 0] - 1
    padded = ((counts + BK - 1) // BK) * BK
    seg_end = jnp.cumsum(padded)
    seg_start = seg_end - padded
    pos = seg_start[e_flat] + rank
    blk_expert = jnp.searchsorted(seg_end, jnp.arange(NB, dtype=jnp.int32) * BK,
                                  side="right").astype(jnp.int32)
    blk_expert = jnp.minimum(blk_expert, E - 1)
    posk = pos.reshape(T, K)
    p0 = posk[:, 0]
    p1 = posk[:, 1]
    g0b = jnp.broadcast_to(g1[:, None], (T, 16))
    g1b = jnp.broadcast_to(g2[:, None], (T, 16))

    xs = _dispatch(x, p0.reshape(NW * _XNC, _XCH), p1.reshape(NW * _XNC, _XCH))
    ys = _mlp(blk_expert, xs, w_in, w_out.astype(jnp.bfloat16))
    out = _combine(ys, p0.reshape(NW * _CNC, _CCH), p1.reshape(NW * _CNC, _CCH),
                   g0b, g1b)
    return out.reshape(1, T, D)


# speedup vs baseline: 1.4517x; 1.4517x over previous
import functools

import jax
import jax.numpy as jnp
from jax import lax
from jax.experimental import pallas as pl
from jax.experimental.pallas import tpu as pltpu
from jax.experimental.pallas import tpu_sc as plsc

E = 8
K = 2
D = 1024
H = 1024
T = 2048
TK = T * K
BK = 256
R = 6144
NB = R // BK

NC = 2
NS = 16
NW = NC * NS


def _router_body(x_ref, rw_ref, i1_ref, i2_ref, g1_ref, g2_ref):
    x = x_ref[...]
    rw = rw_ref[...]
    logits = lax.dot_general(x, rw, (((1,), (1,)), ((), ())),
                             preferred_element_type=jnp.float32)
    cols = lax.broadcasted_iota(jnp.int32, logits.shape, 1)
    i1 = jnp.argmax(logits, axis=1).astype(jnp.int32)
    m1 = jnp.max(logits, axis=1)
    masked = jnp.where(cols == i1[:, None], -jnp.inf, logits)
    i2 = jnp.argmax(masked, axis=1).astype(jnp.int32)
    m2 = jnp.max(masked, axis=1)
    g1 = 1.0 / (1.0 + jnp.exp(m2 - m1))
    i1_ref[...] = i1
    i2_ref[...] = i2
    g1_ref[...] = g1
    g2_ref[...] = 1.0 - g1


def _router(x, router_w):
    return pl.pallas_call(
        _router_body,
        out_shape=(
            jax.ShapeDtypeStruct((T,), jnp.int32),
            jax.ShapeDtypeStruct((T,), jnp.int32),
            jax.ShapeDtypeStruct((T,), jnp.float32),
            jax.ShapeDtypeStruct((T,), jnp.float32),
        ),
    )(x, router_w)


_XTW = T // NW
_XCH = 32
_XNC = _XTW // _XCH


def _dispatch_body(x_hbm, p0_hbm, p1_hbm, xs_hbm, i0_v, i1_v, buf0_v, buf1_v,
                   g0, g1, s0, s1):
    wid = lax.axis_index("s") * NC + lax.axis_index("c")
    base = wid * _XTW
    pltpu.sync_copy(p0_hbm.at[pl.ds(wid * _XNC, _XNC)], i0_v)
    pltpu.sync_copy(p1_hbm.at[pl.ds(wid * _XNC, _XNC)], i1_v)
    bufs = (buf0_v, buf1_v)
    gsem = (g0, g1)
    ssem = (s0, s1)
    rd = [pltpu.make_async_copy(x_hbm.at[pl.ds(base + j * _XCH, _XCH)],
                                bufs[j], gsem[j])
          for j in range(_XNC)]
    w0 = [pltpu.make_async_copy(bufs[j], xs_hbm.at[i0_v.at[j]], ssem[j])
          for j in range(_XNC)]
    w1 = [pltpu.make_async_copy(bufs[j], xs_hbm.at[i1_v.at[j]], ssem[j])
          for j in range(_XNC)]
    for j in range(_XNC):
        rd[j].start()
    for j in range(_XNC):
        rd[j].wait()
        w0[j].start()
        w1[j].start()
    for j in range(_XNC):
        w0[j].wait()
        w1[j].wait()


def _dispatch(x, p0, p1):
    mesh = plsc.VectorSubcoreMesh(core_axis_name="c", subcore_axis_name="s")
    return pl.kernel(
        _dispatch_body,
        out_type=jax.ShapeDtypeStruct((R, D), jnp.float32),
        mesh=mesh,
        scratch_types=[
            pltpu.VMEM((_XNC, _XCH), jnp.int32),
            pltpu.VMEM((_XNC, _XCH), jnp.int32),
            pltpu.VMEM((_XCH, D), jnp.float32),
            pltpu.VMEM((_XCH, D), jnp.float32),
            pltpu.SemaphoreType.DMA,
            pltpu.SemaphoreType.DMA,
            pltpu.SemaphoreType.DMA,
            pltpu.SemaphoreType.DMA,
        ],
    )(x, p0, p1)


_KS = 2
_HC = 2 * H // _KS


def _mlp_body(be_ref, x_ref, wi_ref, wo_ref, o_ref, h_acc):
    k = pl.program_id(1)
    x = x_ref[...]
    part = lax.dot_general(x, wi_ref[0], (((1,), (1,)), ((), ())),
                           precision=lax.Precision.DEFAULT,
                           preferred_element_type=jnp.float32)
    h_acc[:, pl.ds(k * _HC, _HC)] = part

    @pl.when(k == _KS - 1)
    def _():
        h = h_acc[...]
        left = h[:, :H]
        right = h[:, H:]
        a = left * jax.nn.sigmoid(left) * right
        wo = wo_ref[0]
        y = lax.dot_general(a.astype(jnp.bfloat16), wo, (((1,), (1,)), ((), ())),
                            precision=lax.Precision.DEFAULT,
                            preferred_element_type=jnp.float32)
        o_ref[...] = y


def _mlp(blk_expert, xs, w_in, w_out_bf):
    grid_spec = pltpu.PrefetchScalarGridSpec(
        num_scalar_prefetch=1,
        grid=(NB, _KS),
        in_specs=[
            pl.BlockSpec((BK, D), lambda b, k, be: (b, 0)),
            pl.BlockSpec((1, _HC, D), lambda b, k, be: (be[b], k, 0)),
            pl.BlockSpec((1, D, H), lambda b, k, be: (be[b], 0, 0)),
        ],
        out_specs=pl.BlockSpec((BK, D), lambda b, k, be: (b, 0)),
        scratch_shapes=[pltpu.VMEM((BK, 2 * H), jnp.float32)],
    )
    return pl.pallas_call(
        _mlp_body,
        grid_spec=grid_spec,
        out_shape=jax.ShapeDtypeStruct((R, D), jnp.float32),
        compiler_params=pltpu.CompilerParams(
            dimension_semantics=("arbitrary", "arbitrary")),
    )(blk_expert, xs, w_in, w_out_bf)


_TPW = T // NW
_CCH = 16
_CNC = _TPW // _CCH


def _combine_body(ys_hbm, p0_hbm, p1_hbm, g0b_hbm, g1b_hbm, out_hbm, i0_v, i1_v,
                  g0v, g1v, a0_v, b0_v, a1_v, b1_v, g0, g1, s0, s1):
    wid = lax.axis_index("s") * NC + lax.axis_index("c")
    base = wid * _TPW
    pltpu.sync_copy(p0_hbm.at[pl.ds(wid * _CNC, _CNC)], i0_v)
    pltpu.sync_copy(p1_hbm.at[pl.ds(wid * _CNC, _CNC)], i1_v)
    pltpu.sync_copy(g0b_hbm.at[pl.ds(base, _TPW)], g0v)
    pltpu.sync_copy(g1b_hbm.at[pl.ds(base, _TPW)], g1v)
    av = (a0_v, a1_v)
    bv = (b0_v, b1_v)
    gsem = (g0, g1)
    ssem = (s0, s1)
    ga = [pltpu.make_async_copy(ys_hbm.at[i0_v.at[j]], av[j & 1], gsem[j & 1])
          for j in range(_CNC)]
    gb = [pltpu.make_async_copy(ys_hbm.at[i1_v.at[j]], bv[j & 1], gsem[j & 1])
          for j in range(_CNC)]
    scp = [pltpu.make_async_copy(av[j & 1],
                                 out_hbm.at[pl.ds(base + j * _CCH, _CCH)],
                                 ssem[j & 1])
           for j in range(_CNC)]
    ga[0].start()
    gb[0].start()
    for j in range(_CNC):
        if j + 1 < _CNC:
            if j >= 1:
                scp[j - 1].wait()
            ga[j + 1].start()
            gb[j + 1].start()
        ga[j].wait()
        gb[j].wait()
        a_v = av[j & 1]
        b_v = bv[j & 1]

        jj = j * _CCH

        @plsc.parallel_loop(0, _CCH * D, 16, unroll=8)
        def _add(f):
            row = lax.shift_right_logical(f, 10)
            col = pl.multiple_of(lax.bitwise_and(f, D - 1), 16)
            sl = pl.ds(col, 16)
            tr = jj + row
            a_v[row, sl] = (a_v[row, sl] * g0v[tr, :]
                            + b_v[row, sl] * g1v[tr, :])

        scp[j].start()
    scp[_CNC - 2].wait()
    scp[_CNC - 1].wait()


def _combine(ys, p0, p1, g0b, g1b):
    mesh = plsc.VectorSubcoreMesh(core_axis_name="c", subcore_axis_name="s")
    return pl.kernel(
        _combine_body,
        out_type=jax.ShapeDtypeStruct((T, D), jnp.float32),
        mesh=mesh,
        scratch_types=[
            pltpu.VMEM((_CNC, _CCH), jnp.int32),
            pltpu.VMEM((_CNC, _CCH), jnp.int32),
            pltpu.VMEM((_TPW, 16), jnp.float32),
            pltpu.VMEM((_TPW, 16), jnp.float32),
            pltpu.VMEM((_CCH, D), jnp.float32),
            pltpu.VMEM((_CCH, D), jnp.float32),
            pltpu.VMEM((_CCH, D), jnp.float32),
            pltpu.VMEM((_CCH, D), jnp.float32),
            pltpu.SemaphoreType.DMA,
            pltpu.SemaphoreType.DMA,
            pltpu.SemaphoreType.DMA,
            pltpu.SemaphoreType.DMA,
        ],
    )(ys, p0, p1, g0b, g1b)


def kernel(layer_input, router_w, w_in, w_out):
    x = layer_input.reshape(T, D)
    i1, i2, g1, g2 = _router(x, router_w)

    e_flat = jnp.stack([i1, i2], axis=1).reshape(-1)
    oh = (e_flat[:, None] == jnp.arange(E, dtype=jnp.int32)[None, :]).astype(jnp.int32)
    csum = jnp.cumsum(oh, axis=0)
    counts = csum[-1]
    rank = jnp.take_along_axis(csum, e_flat[:, None], axis=1)[:, 0] - 1
    padded = ((counts + BK - 1) // BK) * BK
    seg_end = jnp.cumsum(padded)
    seg_start = seg_end - padded
    pos = seg_start[e_flat] + rank
    blk_expert = jnp.searchsorted(seg_end, jnp.arange(NB, dtype=jnp.int32) * BK,
                                  side="right").astype(jnp.int32)
    blk_expert = jnp.minimum(blk_expert, E - 1)
    posk = pos.reshape(T, K)
    p0 = posk[:, 0]
    p1 = posk[:, 1]
    g0b = jnp.broadcast_to(g1[:, None], (T, 16))
    g1b = jnp.broadcast_to(g2[:, None], (T, 16))

    xs = _dispatch(x, p0.reshape(NW * _XNC, _XCH), p1.reshape(NW * _XNC, _XCH))
    ys = _mlp(blk_expert, xs, w_in, w_out.astype(jnp.bfloat16))
    out = _combine(ys, p0.reshape(NW * _CNC, _CCH), p1.reshape(NW * _CNC, _CCH),
                   g0b, g1b)
    return out.reshape(1, T, D)


# speedup vs baseline: 1.9450x; 1.3398x over previous
import functools

import jax
import jax.numpy as jnp
from jax import lax
from jax.experimental import pallas as pl
from jax.experimental.pallas import tpu as pltpu
from jax.experimental.pallas import tpu_sc as plsc

E = 8
K = 2
D = 1024
H = 1024
T = 2048
TK = T * K
BK = 256
R = 6144
NB = R // BK

NC = 2
NS = 16
NW = NC * NS


_PC = 128
_PNC = TK // _PC


def _router_body(x_ref, rw_ref, pos_ref, be_ref, g1_ref, g2_ref, oh_s):
    x = x_ref[...]
    rw = rw_ref[...]
    logits = lax.dot_general(rw, x, (((1,), (1,)), ((), ())),
                             preferred_element_type=jnp.float32)
    erange = lax.broadcasted_iota(jnp.int32, (E, T), 0)
    i1 = jnp.argmax(logits, axis=0).astype(jnp.int32)
    m1 = jnp.max(logits, axis=0)
    masked = jnp.where(erange == i1[None, :], -jnp.inf, logits)
    i2 = jnp.argmax(masked, axis=0).astype(jnp.int32)
    m2 = jnp.max(masked, axis=0)
    g1 = 1.0 / (1.0 + jnp.exp(m2 - m1))
    g1_ref[...] = g1
    g2_ref[...] = 1.0 - g1

    oh1 = (erange == i1[None, :]).astype(jnp.float32)
    oh2 = (erange == i2[None, :]).astype(jnp.float32)
    oh_s[...] = jnp.concatenate([oh1, oh2], axis=1)

    counts = jnp.sum(oh1, axis=1) + jnp.sum(oh2, axis=1)
    padded = jnp.ceil(counts * (1.0 / BK)) * BK
    eia = lax.broadcasted_iota(jnp.int32, (E, E), 0)
    eib = lax.broadcasted_iota(jnp.int32, (E, E), 1)
    u8 = (eia <= eib).astype(jnp.float32)
    seg_end = lax.dot_general(padded.reshape(1, E), u8,
                              (((1,), (0,)), ((), ())),
                              preferred_element_type=jnp.float32).reshape(E)
    seg_start = seg_end - padded

    bia = lax.broadcasted_iota(jnp.int32, (_PC, _PC), 0)
    bib = lax.broadcasted_iota(jnp.int32, (_PC, _PC), 1)
    u128 = (bia <= bib).astype(jnp.float32)
    seg_start_b = jnp.broadcast_to(seg_start[:, None], (E, _PC))

    def chunk(c, run):
        ohc = oh_s[:, pl.ds(c * _PC, _PC)]
        csum = lax.dot_general(ohc, u128, (((1,), (0,)), ((), ())),
                               preferred_element_type=jnp.float32)
        csum = csum + run[:, None]
        posv = jnp.sum(ohc * (seg_start_b + csum), axis=0) - 1.0
        pos_ref[pl.ds(c * _PC, _PC)] = posv.astype(jnp.int32)
        return run + jnp.sum(ohc, axis=1)

    lax.fori_loop(0, _PNC, chunk, jnp.zeros((E,), jnp.float32))

    bvals = lax.broadcasted_iota(jnp.int32, (E, NB), 1).astype(jnp.float32) * float(BK)
    seg_end_b = jnp.broadcast_to(seg_end[:, None], (E, NB))
    be = jnp.sum((seg_end_b <= bvals).astype(jnp.int32), axis=0)
    be_ref[...] = jnp.minimum(be, E - 1)


def _router(x, router_w):
    return pl.pallas_call(
        _router_body,
        out_shape=(
            jax.ShapeDtypeStruct((TK,), jnp.int32),
            jax.ShapeDtypeStruct((NB,), jnp.int32),
            jax.ShapeDtypeStruct((T,), jnp.float32),
            jax.ShapeDtypeStruct((T,), jnp.float32),
        ),
        scratch_shapes=[pltpu.VMEM((E, TK), jnp.float32)],
    )(x, router_w)


_XTW = T // NW
_XCH = 32
_XNC = _XTW // _XCH


def _dispatch_body(x_hbm, p0_hbm, p1_hbm, xs_hbm, i0_v, i1_v, buf0_v, buf1_v,
                   g0, g1, s0, s1):
    wid = lax.axis_index("s") * NC + lax.axis_index("c")
    base = wid * _XTW
    pltpu.sync_copy(p0_hbm.at[pl.ds(wid * _XNC, _XNC)], i0_v)
    pltpu.sync_copy(p1_hbm.at[pl.ds(wid * _XNC, _XNC)], i1_v)
    bufs = (buf0_v, buf1_v)
    gsem = (g0, g1)
    ssem = (s0, s1)
    rd = [pltpu.make_async_copy(x_hbm.at[pl.ds(base + j * _XCH, _XCH)],
                                bufs[j], gsem[j])
          for j in range(_XNC)]
    w0 = [pltpu.make_async_copy(bufs[j], xs_hbm.at[i0_v.at[j]], ssem[j])
          for j in range(_XNC)]
    w1 = [pltpu.make_async_copy(bufs[j], xs_hbm.at[i1_v.at[j]], ssem[j])
          for j in range(_XNC)]
    for j in range(_XNC):
        rd[j].start()
    for j in range(_XNC):
        rd[j].wait()
        w0[j].start()
        w1[j].start()
    for j in range(_XNC):
        w0[j].wait()
        w1[j].wait()


def _dispatch(x, p0, p1):
    mesh = plsc.VectorSubcoreMesh(core_axis_name="c", subcore_axis_name="s")
    return pl.kernel(
        _dispatch_body,
        out_type=jax.ShapeDtypeStruct((R, D), jnp.float32),
        mesh=mesh,
        scratch_types=[
            pltpu.VMEM((_XNC, _XCH), jnp.int32),
            pltpu.VMEM((_XNC, _XCH), jnp.int32),
            pltpu.VMEM((_XCH, D), jnp.float32),
            pltpu.VMEM((_XCH, D), jnp.float32),
            pltpu.SemaphoreType.DMA,
            pltpu.SemaphoreType.DMA,
            pltpu.SemaphoreType.DMA,
            pltpu.SemaphoreType.DMA,
        ],
    )(x, p0, p1)


def _mlp_body(be_ref, x_ref, wi_ref, wo_ref, o_ref):
    x = x_ref[...]
    h = lax.dot_general(x, wi_ref[0], (((1,), (1,)), ((), ())),
                        precision=lax.Precision.DEFAULT,
                        preferred_element_type=jnp.float32)
    left = h[:, :H]
    right = h[:, H:]
    a = left * jax.nn.sigmoid(left) * right
    wo = wo_ref[0]
    y = lax.dot_general(a.astype(jnp.bfloat16), wo, (((1,), (1,)), ((), ())),
                        precision=lax.Precision.DEFAULT,
                        preferred_element_type=jnp.float32)
    o_ref[...] = y


def _mlp(blk_expert, xs, w_in, w_out_bf):
    grid_spec = pltpu.PrefetchScalarGridSpec(
        num_scalar_prefetch=1,
        grid=(NB,),
        in_specs=[
            pl.BlockSpec((BK, D), lambda b, be: (b, 0)),
            pl.BlockSpec((1, 2 * H, D), lambda b, be: (be[b], 0, 0)),
            pl.BlockSpec((1, D, H), lambda b, be: (be[b], 0, 0)),
        ],
        out_specs=pl.BlockSpec((BK, D), lambda b, be: (b, 0)),
    )
    return pl.pallas_call(
        _mlp_body,
        grid_spec=grid_spec,
        out_shape=jax.ShapeDtypeStruct((R, D), jnp.float32),
    )(blk_expert, xs, w_in, w_out_bf)


_TPW = T // NW
_CCH = 16
_CNC = _TPW // _CCH


def _combine_body(ys_hbm, p0_hbm, p1_hbm, g0b_hbm, g1b_hbm, out_hbm, i0_v, i1_v,
                  g0v, g1v, a0_v, b0_v, a1_v, b1_v, g0, g1, s0, s1):
    wid = lax.axis_index("s") * NC + lax.axis_index("c")
    base = wid * _TPW
    pltpu.sync_copy(p0_hbm.at[pl.ds(wid * _CNC, _CNC)], i0_v)
    pltpu.sync_copy(p1_hbm.at[pl.ds(wid * _CNC, _CNC)], i1_v)
    pltpu.sync_copy(g0b_hbm.at[pl.ds(base, _TPW)], g0v)
    pltpu.sync_copy(g1b_hbm.at[pl.ds(base, _TPW)], g1v)
    av = (a0_v, a1_v)
    bv = (b0_v, b1_v)
    gsem = (g0, g1)
    ssem = (s0, s1)
    ga = [pltpu.make_async_copy(ys_hbm.at[i0_v.at[j]], av[j & 1], gsem[j & 1])
          for j in range(_CNC)]
    gb = [pltpu.make_async_copy(ys_hbm.at[i1_v.at[j]], bv[j & 1], gsem[j & 1])
          for j in range(_CNC)]
    scp = [pltpu.make_async_copy(av[j & 1],
                                 out_hbm.at[pl.ds(base + j * _CCH, _CCH)],
                                 ssem[j & 1])
           for j in range(_CNC)]
    ga[0].start()
    gb[0].start()
    for j in range(_CNC):
        if j + 1 < _CNC:
            if j >= 1:
                scp[j - 1].wait()
            ga[j + 1].start()
            gb[j + 1].start()
        ga[j].wait()
        gb[j].wait()
        a_v = av[j & 1]
        b_v = bv[j & 1]

        jj = j * _CCH

        @plsc.parallel_loop(0, _CCH * D, 16, unroll=8)
        def _add(f):
            row = lax.shift_right_logical(f, 10)
            col = pl.multiple_of(lax.bitwise_and(f, D - 1), 16)
            sl = pl.ds(col, 16)
            tr = jj + row
            a_v[row, sl] = (a_v[row, sl] * g0v[tr, :]
                            + b_v[row, sl] * g1v[tr, :])

        scp[j].start()
    scp[_CNC - 2].wait()
    scp[_CNC - 1].wait()


def _combine(ys, p0, p1, g0b, g1b):
    mesh = plsc.VectorSubcoreMesh(core_axis_name="c", subcore_axis_name="s")
    return pl.kernel(
        _combine_body,
        out_type=jax.ShapeDtypeStruct((T, D), jnp.float32),
        mesh=mesh,
        scratch_types=[
            pltpu.VMEM((_CNC, _CCH), jnp.int32),
            pltpu.VMEM((_CNC, _CCH), jnp.int32),
            pltpu.VMEM((_TPW, 16), jnp.float32),
            pltpu.VMEM((_TPW, 16), jnp.float32),
            pltpu.VMEM((_CCH, D), jnp.float32),
            pltpu.VMEM((_CCH, D), jnp.float32),
            pltpu.VMEM((_CCH, D), jnp.float32),
            pltpu.VMEM((_CCH, D), jnp.float32),
            pltpu.SemaphoreType.DMA,
            pltpu.SemaphoreType.DMA,
            pltpu.SemaphoreType.DMA,
            pltpu.SemaphoreType.DMA,
        ],
    )(ys, p0, p1, g0b, g1b)


def kernel(layer_input, router_w, w_in, w_out):
    x = layer_input.reshape(T, D)
    pos, blk_expert, g1, g2 = _router(x, router_w)
    p0 = pos[:T]
    p1 = pos[T:]
    g0b = jnp.broadcast_to(g1[:, None], (T, 16))
    g1b = jnp.broadcast_to(g2[:, None], (T, 16))

    xs = _dispatch(x, p0.reshape(NW * _XNC, _XCH), p1.reshape(NW * _XNC, _XCH))
    ys = _mlp(blk_expert, xs, w_in, w_out.astype(jnp.bfloat16))
    out = _combine(ys, p0.reshape(NW * _CNC, _CCH), p1.reshape(NW * _CNC, _CCH),
                   g0b, g1b)
    return out.reshape(1, T, D)
